# baseline (speedup 1.0000x reference)
import jax
import jax.numpy as jnp
from jax import lax
from jax.experimental import pallas as pl
from jax.experimental.pallas import tpu as pltpu
from jax.experimental.pallas import tpu_sc as plsc

N = 10000
NP = 10240
E = 320000
D = 128
NT = 16
NC = 2
EPW = E // (NC * NT)
CH = 80
NCH = EPW // CH
EPT = E // NT
NCHL = EPT // CH
RPT = NP // NT
LANES = 16


def _agg_body(src_t, dst_t, xin,
              agg2,
              src_v, dst_v, rows0, rows1, agg_s,
              sem0, sem1, sem2, sem3):
    c = lax.axis_index("c")
    s = lax.axis_index("s")
    t = c * NT + s

    pltpu.async_copy(src_t.at[t], src_v, sem0)
    pltpu.async_copy(dst_t.at[t], dst_v, sem1)

    zeros16 = jnp.zeros((LANES,), jnp.float32)

    @pl.loop(0, CH)
    def _(r):
        for q in range(D // LANES):
            rows0[r, pl.ds(q * LANES, LANES)] = zeros16

    for k in range(RPT // CH):
        pltpu.async_copy(rows0, agg_s.at[pl.ds(s * RPT + k * CH, CH)], sem2)

    pltpu.make_async_copy(src_t.at[t], src_v, sem0).wait()
    pltpu.make_async_copy(dst_t.at[t], dst_v, sem1).wait()
    for k in range(RPT // CH):
        pltpu.make_async_copy(rows0,
                              agg_s.at[pl.ds(s * RPT + k * CH, CH)],
                              sem2).wait()

    plsc.subcore_barrier()

    def g_start(r, buf, sem):
        pltpu.async_copy(xin.at[src_v.at[pl.ds(r * CH, CH)]], buf, sem)

    def g_wait(r, buf, sem):
        pltpu.make_async_copy(xin.at[src_v.at[pl.ds(r * CH, CH)]], buf,
                              sem).wait()

    def s_start(r, buf, sem):
        pltpu.async_copy(buf, agg_s.at[dst_v.at[r]], sem, add=True)

    def s_wait(r, buf, sem):
        pltpu.make_async_copy(buf, agg_s.at[dst_v.at[r]], sem).wait()

    g_start(0, rows0, sem0)
    g_start(1, rows1, sem1)

    @pl.loop(0, NCH // 2)
    def _(r2):
        r = r2 * 2
        g_wait(r, rows0, sem0)
        s_start(r, rows0, sem2)
        g_wait(r + 1, rows1, sem1)
        s_start(r + 1, rows1, sem3)
        s_wait(r, rows0, sem2)

        @pl.when(r + 2 < NCH)
        def _():
            g_start(r + 2, rows0, sem0)

        s_wait(r + 1, rows1, sem3)

        @pl.when(r + 3 < NCH)
        def _():
            g_start(r + 3, rows1, sem1)

    if NCH % 2 == 1:
        r_last = NCH - 1
        g_wait(r_last, rows0, sem0)
        s_start(r_last, rows0, sem2)
        s_wait(r_last, rows0, sem2)

    plsc.subcore_barrier()

    pltpu.sync_copy(agg_s.at[pl.ds(s * RPT, RPT)],
                    agg2.at[c, pl.ds(s * RPT, RPT)])


_agg_call = pl.kernel(
    _agg_body,
    out_type=[
        jax.ShapeDtypeStruct((NC, NP, D), jnp.float32),
    ],
    mesh=plsc.VectorSubcoreMesh(core_axis_name="c", subcore_axis_name="s"),
    compiler_params=pltpu.CompilerParams(needs_layout_passes=False),
    scratch_types=[
        pltpu.VMEM((EPW,), jnp.int32),
        pltpu.VMEM((NCH, CH), jnp.int32),
        pltpu.VMEM((CH, D), jnp.float32),
        pltpu.VMEM((CH, D), jnp.float32),
        pltpu.VMEM_SHARED((NP, D), jnp.float32),
        pltpu.SemaphoreType.DMA,
        pltpu.SemaphoreType.DMA,
        pltpu.SemaphoreType.DMA,
        pltpu.SemaphoreType.DMA,
    ],
)


def _cnt_body(dst_t, cnt32,
              dstf_v, cntp_v):
    c = lax.axis_index("c")
    s = lax.axis_index("s")
    t = c * NT + s

    pltpu.sync_copy(dst_t.at[t], dstf_v)

    zeros16 = jnp.zeros((LANES,), jnp.float32)
    ones16 = jnp.ones((LANES,), jnp.float32)

    @pl.loop(0, NP // LANES)
    def _(i):
        cntp_v[pl.ds(i * LANES, LANES)] = zeros16

    @pl.loop(0, NCH)
    def _(r):
        for q in range(CH // LANES):
            d = dstf_v[r, pl.ds(q * LANES, LANES)]
            plsc.addupdate_scatter(cntp_v, [d], ones16)

    pltpu.sync_copy(cntp_v, cnt32.at[t])


_cnt_call = pl.kernel(
    _cnt_body,
    out_type=[
        jax.ShapeDtypeStruct((NC * NT, NP), jnp.float32),
    ],
    mesh=plsc.VectorSubcoreMesh(core_axis_name="c", subcore_axis_name="s"),
    compiler_params=pltpu.CompilerParams(needs_layout_passes=False),
    scratch_types=[
        pltpu.VMEM((NCH, CH), jnp.int32),
        pltpu.VMEM((NP,), jnp.float32),
    ],
)


BN = 640


def _mm_body(x, agg, cnt, w1l, w1r, b1, w2l, w2r, b2, t_out, rb_out, cc_out):
    cc = jnp.maximum(jnp.sum(cnt[...], axis=0), 1.0)
    recip = 1.0 / cc
    mean = (agg[0] + agg[1]) * recip
    h = (jnp.dot(mean, w1l[...], preferred_element_type=jnp.float32)
         + jnp.dot(x[...], w1r[...], preferred_element_type=jnp.float32)
         + b1[...])
    h = jnp.maximum(h, 0.0)
    t_out[...] = jnp.dot(h, w2l[...], preferred_element_type=jnp.float32)
    rb_out[...] = (jnp.dot(h, w2r[...], preferred_element_type=jnp.float32)
                   + b2[...])
    cc_out[...] = cc


_mm_call = pl.pallas_call(
    _mm_body,
    grid=(NP // BN,),
    in_specs=[
        pl.BlockSpec((BN, D), lambda i: (i, 0)),
        pl.BlockSpec((NC, BN, D), lambda i: (0, i, 0)),
        pl.BlockSpec((NC * NT, BN, 1), lambda i: (0, i, 0)),
        pl.BlockSpec((D, D), lambda i: (0, 0)),
        pl.BlockSpec((D, D), lambda i: (0, 0)),
        pl.BlockSpec((1, D), lambda i: (0, 0)),
        pl.BlockSpec((D, 1), lambda i: (0, 0)),
        pl.BlockSpec((D, 1), lambda i: (0, 0)),
        pl.BlockSpec((1, 1), lambda i: (0, 0)),
    ],
    out_specs=[
        pl.BlockSpec((BN, 1), lambda i: (i, 0)),
        pl.BlockSpec((BN, 1), lambda i: (i, 0)),
        pl.BlockSpec((BN, 1), lambda i: (i, 0)),
    ],
    out_shape=[
        jax.ShapeDtypeStruct((NP, 1), jnp.float32),
        jax.ShapeDtypeStruct((NP, 1), jnp.float32),
        jax.ShapeDtypeStruct((NP, 1), jnp.float32),
    ],
)


def _l2_body(src_l, dst_l, t_in, rb_in, cc_in,
             out,
             t_v, src_v, dst_v, acc_v, part_v, cc_v, rb_v, out_v,
             part_s, lsem0, lsem1, lsem2):
    c = lax.axis_index("c")
    s = lax.axis_index("s")

    @pl.when(c == 0)
    def _():
        pltpu.async_copy(t_in, t_v, lsem0)
        pltpu.async_copy(src_l.at[s], src_v, lsem1)
        pltpu.async_copy(dst_l.at[s], dst_v, lsem2)

        zeros16 = jnp.zeros((LANES,), jnp.float32)

        @pl.loop(0, NP // LANES)
        def _(i):
            acc_v[pl.ds(i * LANES, LANES)] = zeros16

        pltpu.make_async_copy(t_in, t_v, lsem0).wait()
        pltpu.make_async_copy(src_l.at[s], src_v, lsem1).wait()
        pltpu.make_async_copy(dst_l.at[s], dst_v, lsem2).wait()

        @pl.loop(0, NCHL)
        def _(r):
            for q in range(CH // LANES):
                sl = pl.ds(q * LANES, LANES)
                vals = plsc.load_gather(t_v, [src_v[r, sl]])
                plsc.addupdate_scatter(acc_v, [dst_v[r, sl]], vals)

        pltpu.sync_copy(acc_v, part_s.at[s])
        plsc.subcore_barrier()

        for k in range(NT):
            pltpu.sync_copy(part_s.at[k, pl.ds(s * RPT, RPT)], part_v.at[k])
        pltpu.sync_copy(cc_in.at[pl.ds(s * RPT, RPT)], cc_v)
        pltpu.sync_copy(rb_in.at[pl.ds(s * RPT, RPT)], rb_v)

        @pl.loop(0, RPT // LANES)
        def _(q):
            sl = pl.ds(q * LANES, LANES)
            acc = part_v[0, sl]
            for k in range(1, NT):
                acc = acc + part_v[k, sl]
            out_v[sl] = acc / cc_v[sl] + rb_v[sl]

        pltpu.sync_copy(out_v, out.at[pl.ds(s * RPT, RPT)])


_l2_call = pl.kernel(
    _l2_body,
    out_type=[jax.ShapeDtypeStruct((NP,), jnp.float32)],
    mesh=plsc.VectorSubcoreMesh(core_axis_name="c", subcore_axis_name="s"),
    compiler_params=pltpu.CompilerParams(needs_layout_passes=False),
    scratch_types=[
        pltpu.VMEM((NP,), jnp.float32),
        pltpu.VMEM((NCHL, CH), jnp.int32),
        pltpu.VMEM((NCHL, CH), jnp.int32),
        pltpu.VMEM((NP,), jnp.float32),
        pltpu.VMEM((NT, RPT), jnp.float32),
        pltpu.VMEM((RPT,), jnp.float32),
        pltpu.VMEM((RPT,), jnp.float32),
        pltpu.VMEM((RPT,), jnp.float32),
        pltpu.VMEM_SHARED((NT, NP), jnp.float32),
        pltpu.SemaphoreType.DMA,
        pltpu.SemaphoreType.DMA,
        pltpu.SemaphoreType.DMA,
    ],
)


def kernel(x, edge_index, W1_l, W1_r, b1, W2_l, W2_r, b2):
    src = edge_index[0].astype(jnp.int32)
    dst = edge_index[1].astype(jnp.int32)

    src_t = src.reshape(NC * NT, EPW)
    dst_t = dst.reshape(NC * NT, NCH, CH)

    src_l = src.reshape(NT, NCHL, CH)
    dst_l = dst.reshape(NT, NCHL, CH)

    (cnt32,) = _cnt_call(dst_t)
    (agg2,) = _agg_call(src_t, dst_t, x)

    xp = jnp.pad(x, ((0, NP - N), (0, 0)))
    t, rb, cc = _mm_call(xp, agg2.reshape(NC, NP, D),
                         cnt32.reshape(NC * NT, NP, 1),
                         W1_l, W1_r, b1.reshape(1, D), W2_l, W2_r,
                         b2.reshape(1, 1))

    (out,) = _l2_call(src_l, dst_l,
                      t.reshape(NP), rb.reshape(NP), cc.reshape(NP))
    return out[:N]


# speedup vs baseline: 1.1835x; 1.1835x over previous
import jax
import jax.numpy as jnp
from jax import lax
from jax.experimental import pallas as pl
from jax.experimental.pallas import tpu as pltpu
from jax.experimental.pallas import tpu_sc as plsc

N = 10000
NP = 10240
E = 320000
D = 128
NT = 16
NC = 2
EPW = E // (NC * NT)
CH = 80
NCH = EPW // CH
EPT = E // NT
NCHL = EPT // CH
RPT = NP // NT
LANES = 16


def _agg_body(src_t, dst_t, xin,
              agg2,
              src_v, dst_v, rows0, rows1, agg_s,
              sem0, sem1, sem2, sem3):
    c = lax.axis_index("c")
    s = lax.axis_index("s")
    t = c * NT + s

    pltpu.async_copy(src_t.at[t], src_v, sem0)
    pltpu.async_copy(dst_t.at[t], dst_v, sem1)

    zeros16 = jnp.zeros((LANES,), jnp.float32)

    @pl.loop(0, CH)
    def _(r):
        for q in range(D // LANES):
            rows0[r, pl.ds(q * LANES, LANES)] = zeros16

    for k in range(RPT // CH):
        pltpu.async_copy(rows0, agg_s.at[pl.ds(s * RPT + k * CH, CH)], sem2)

    pltpu.make_async_copy(src_t.at[t], src_v, sem0).wait()
    pltpu.make_async_copy(dst_t.at[t], dst_v, sem1).wait()
    for k in range(RPT // CH):
        pltpu.make_async_copy(rows0,
                              agg_s.at[pl.ds(s * RPT + k * CH, CH)],
                              sem2).wait()

    plsc.subcore_barrier()

    def g_start(r, buf, sem):
        pltpu.async_copy(xin.at[src_v.at[pl.ds(r * CH, CH)]], buf, sem)

    def g_wait(r, buf, sem):
        pltpu.make_async_copy(xin.at[src_v.at[pl.ds(r * CH, CH)]], buf,
                              sem).wait()

    def s_start(r, buf, sem):
        pltpu.async_copy(buf, agg_s.at[dst_v.at[r]], sem, add=True)

    def s_wait(r, buf, sem):
        pltpu.make_async_copy(buf, agg_s.at[dst_v.at[r]], sem).wait()

    g_start(0, rows0, sem0)
    g_start(1, rows1, sem1)

    @pl.loop(0, NCH // 2)
    def _(r2):
        r = r2 * 2
        g_wait(r, rows0, sem0)
        s_start(r, rows0, sem2)
        g_wait(r + 1, rows1, sem1)
        s_start(r + 1, rows1, sem3)
        s_wait(r, rows0, sem2)

        @pl.when(r + 2 < NCH)
        def _():
            g_start(r + 2, rows0, sem0)

        s_wait(r + 1, rows1, sem3)

        @pl.when(r + 3 < NCH)
        def _():
            g_start(r + 3, rows1, sem1)

    if NCH % 2 == 1:
        r_last = NCH - 1
        g_wait(r_last, rows0, sem0)
        s_start(r_last, rows0, sem2)
        s_wait(r_last, rows0, sem2)

    plsc.subcore_barrier()

    pltpu.sync_copy(agg_s.at[pl.ds(s * RPT, RPT)],
                    agg2.at[c, pl.ds(s * RPT, RPT)])


_agg_call = pl.kernel(
    _agg_body,
    out_type=[
        jax.ShapeDtypeStruct((NC, NP, D), jnp.float32),
    ],
    mesh=plsc.VectorSubcoreMesh(core_axis_name="c", subcore_axis_name="s"),
    compiler_params=pltpu.CompilerParams(needs_layout_passes=False),
    scratch_types=[
        pltpu.VMEM((EPW,), jnp.int32),
        pltpu.VMEM((NCH, CH), jnp.int32),
        pltpu.VMEM((CH, D), jnp.float32),
        pltpu.VMEM((CH, D), jnp.float32),
        pltpu.VMEM_SHARED((NP, D), jnp.float32),
        pltpu.SemaphoreType.DMA,
        pltpu.SemaphoreType.DMA,
        pltpu.SemaphoreType.DMA,
        pltpu.SemaphoreType.DMA,
    ],
)


def _cnt_body(dst_t, cnt2,
              dstf_v, cntp_v, part_v, cnts_v, cstage_s):
    c = lax.axis_index("c")
    s = lax.axis_index("s")
    t = c * NT + s

    pltpu.sync_copy(dst_t.at[t], dstf_v)

    zeros16 = jnp.zeros((LANES,), jnp.float32)
    ones16 = jnp.ones((LANES,), jnp.float32)

    @pl.loop(0, NP // LANES)
    def _(i):
        cntp_v[pl.ds(i * LANES, LANES)] = zeros16

    @pl.loop(0, NCH)
    def _(r):
        for q in range(CH // LANES):
            d = dstf_v[r, pl.ds(q * LANES, LANES)]
            plsc.addupdate_scatter(cntp_v, [d], ones16)

    pltpu.sync_copy(cntp_v, cstage_s.at[s])
    plsc.subcore_barrier()

    for k in range(NT):
        pltpu.sync_copy(cstage_s.at[k, pl.ds(s * RPT, RPT)], part_v.at[k])

    @pl.loop(0, RPT // LANES)
    def _(q):
        sl = pl.ds(q * LANES, LANES)
        acc = part_v[0, sl]
        for k in range(1, NT):
            acc = acc + part_v[k, sl]
        cnts_v[sl] = acc

    pltpu.sync_copy(cnts_v, cnt2.at[c, pl.ds(s * RPT, RPT)])


_cnt_call = pl.kernel(
    _cnt_body,
    out_type=[
        jax.ShapeDtypeStruct((NC, NP), jnp.float32),
    ],
    mesh=plsc.VectorSubcoreMesh(core_axis_name="c", subcore_axis_name="s"),
    compiler_params=pltpu.CompilerParams(needs_layout_passes=False),
    scratch_types=[
        pltpu.VMEM((NCH, CH), jnp.int32),
        pltpu.VMEM((NP,), jnp.float32),
        pltpu.VMEM((NT, RPT), jnp.float32),
        pltpu.VMEM((RPT,), jnp.float32),
        pltpu.VMEM_SHARED((NT, NP), jnp.float32),
    ],
)


BN = 640


def _mm_body(x, agg, cnt, w1l, w1r, b1, w2l, w2r, b2, t_out, rb_out, cc_out):
    cc = jnp.maximum(cnt[0] + cnt[1], 1.0)
    recip = 1.0 / cc
    mean = (agg[0] + agg[1]) * recip
    h = (jnp.dot(mean, w1l[...], preferred_element_type=jnp.float32)
         + jnp.dot(x[...], w1r[...], preferred_element_type=jnp.float32)
         + b1[...])
    h = jnp.maximum(h, 0.0)
    t_out[...] = jnp.dot(h, w2l[...], preferred_element_type=jnp.float32)
    rb_out[...] = (jnp.dot(h, w2r[...], preferred_element_type=jnp.float32)
                   + b2[...])
    cc_out[...] = cc


_mm_call = pl.pallas_call(
    _mm_body,
    grid=(NP // BN,),
    in_specs=[
        pl.BlockSpec((BN, D), lambda i: (i, 0)),
        pl.BlockSpec((NC, BN, D), lambda i: (0, i, 0)),
        pl.BlockSpec((NC, BN, 1), lambda i: (0, i, 0)),
        pl.BlockSpec((D, D), lambda i: (0, 0)),
        pl.BlockSpec((D, D), lambda i: (0, 0)),
        pl.BlockSpec((1, D), lambda i: (0, 0)),
        pl.BlockSpec((D, 1), lambda i: (0, 0)),
        pl.BlockSpec((D, 1), lambda i: (0, 0)),
        pl.BlockSpec((1, 1), lambda i: (0, 0)),
    ],
    out_specs=[
        pl.BlockSpec((BN, 1), lambda i: (i, 0)),
        pl.BlockSpec((BN, 1), lambda i: (i, 0)),
        pl.BlockSpec((BN, 1), lambda i: (i, 0)),
    ],
    out_shape=[
        jax.ShapeDtypeStruct((NP, 1), jnp.float32),
        jax.ShapeDtypeStruct((NP, 1), jnp.float32),
        jax.ShapeDtypeStruct((NP, 1), jnp.float32),
    ],
)


def _l2_body(src_l, dst_l, t_in, rb_in, cc_in,
             out,
             t_v, src_v, dst_v, acc_v, part_v, cc_v, rb_v, out_v,
             part_s, lsem0, lsem1, lsem2):
    c = lax.axis_index("c")
    s = lax.axis_index("s")

    @pl.when(c == 0)
    def _():
        pltpu.async_copy(t_in, t_v, lsem0)
        pltpu.async_copy(src_l.at[s], src_v, lsem1)
        pltpu.async_copy(dst_l.at[s], dst_v, lsem2)

        zeros16 = jnp.zeros((LANES,), jnp.float32)

        @pl.loop(0, NP // LANES)
        def _(i):
            acc_v[pl.ds(i * LANES, LANES)] = zeros16

        pltpu.make_async_copy(t_in, t_v, lsem0).wait()
        pltpu.make_async_copy(src_l.at[s], src_v, lsem1).wait()
        pltpu.make_async_copy(dst_l.at[s], dst_v, lsem2).wait()

        @pl.loop(0, NCHL)
        def _(r):
            for q in range(CH // LANES):
                sl = pl.ds(q * LANES, LANES)
                vals = plsc.load_gather(t_v, [src_v[r, sl]])
                plsc.addupdate_scatter(acc_v, [dst_v[r, sl]], vals)

        pltpu.sync_copy(acc_v, part_s.at[s])
        plsc.subcore_barrier()

        for k in range(NT):
            pltpu.sync_copy(part_s.at[k, pl.ds(s * RPT, RPT)], part_v.at[k])
        pltpu.sync_copy(cc_in.at[pl.ds(s * RPT, RPT)], cc_v)
        pltpu.sync_copy(rb_in.at[pl.ds(s * RPT, RPT)], rb_v)

        @pl.loop(0, RPT // LANES)
        def _(q):
            sl = pl.ds(q * LANES, LANES)
            acc = part_v[0, sl]
            for k in range(1, NT):
                acc = acc + part_v[k, sl]
            out_v[sl] = acc / cc_v[sl] + rb_v[sl]

        pltpu.sync_copy(out_v, out.at[pl.ds(s * RPT, RPT)])


_l2_call = pl.kernel(
    _l2_body,
    out_type=[jax.ShapeDtypeStruct((NP,), jnp.float32)],
    mesh=plsc.VectorSubcoreMesh(core_axis_name="c", subcore_axis_name="s"),
    compiler_params=pltpu.CompilerParams(needs_layout_passes=False),
    scratch_types=[
        pltpu.VMEM((NP,), jnp.float32),
        pltpu.VMEM((NCHL, CH), jnp.int32),
        pltpu.VMEM((NCHL, CH), jnp.int32),
        pltpu.VMEM((NP,), jnp.float32),
        pltpu.VMEM((NT, RPT), jnp.float32),
        pltpu.VMEM((RPT,), jnp.float32),
        pltpu.VMEM((RPT,), jnp.float32),
        pltpu.VMEM((RPT,), jnp.float32),
        pltpu.VMEM_SHARED((NT, NP), jnp.float32),
        pltpu.SemaphoreType.DMA,
        pltpu.SemaphoreType.DMA,
        pltpu.SemaphoreType.DMA,
    ],
)


def kernel(x, edge_index, W1_l, W1_r, b1, W2_l, W2_r, b2):
    src = edge_index[0].astype(jnp.int32)
    dst = edge_index[1].astype(jnp.int32)

    src_t = src.reshape(NC * NT, EPW)
    dst_t = dst.reshape(NC * NT, NCH, CH)

    src_l = src.reshape(NT, NCHL, CH)
    dst_l = dst.reshape(NT, NCHL, CH)

    (cnt2,) = _cnt_call(dst_t)
    (agg2,) = _agg_call(src_t, dst_t, x)

    xp = jnp.pad(x, ((0, NP - N), (0, 0)))
    t, rb, cc = _mm_call(xp, agg2.reshape(NC, NP, D), cnt2.reshape(NC, NP, 1),
                         W1_l, W1_r, b1.reshape(1, D), W2_l, W2_r,
                         b2.reshape(1, 1))

    (out,) = _l2_call(src_l, dst_l,
                      t.reshape(NP), rb.reshape(NP), cc.reshape(NP))
    return out[:N]
